# baseline (speedup 1.0000x reference)
import functools

import jax
import jax.numpy as jnp
from jax import lax
from jax.experimental import pallas as pl
from jax.experimental.pallas import tpu as pltpu
from jax.experimental.pallas import tpu_sc as plsc

_NC, _NS, _L = 2, 16, 16
_NW = _NC * _NS

_BPC = 4
_NBUF = 4
_K = 2


@functools.lru_cache(maxsize=None)
def _build(S, B, D, VS):
    N = B * S
    GR = _BPC * S
    BPW = B // _NW
    RPW = BPW // _BPC
    CW = BPW * S
    PAT = S * _L
    NV = CW // _L
    assert B % (_NW * _BPC) == 0 and RPW % _NBUF == 0 and CW % PAT == 0

    mesh = plsc.VectorSubcoreMesh(core_axis_name="c", subcore_axis_name="s")

    @functools.partial(
        pl.kernel,
        mesh=mesh,
        out_type=jax.ShapeDtypeStruct((B, S, D), jnp.float32),
        compiler_params=pltpu.CompilerParams(
            use_tc_tiling_on_sc=False, needs_layout_passes=False
        ),
        scratch_types=[
            pltpu.VMEM((CW,), jnp.int32),
            pltpu.VMEM((PAT,), jnp.int32),
            pltpu.VMEM((_NBUF, GR, D), jnp.float32),
        ]
        + [pltpu.SemaphoreType.DMA] * (2 * _NBUF),
    )
    def k(table_hbm, in_hbm, pat_hbm, out_hbm, idx_v, pat_v, rows_v, *sems):
        gsem, wsem = sems[:_NBUF], sems[_NBUF:]
        wid = lax.axis_index("s") * _NC + lax.axis_index("c")
        b0 = wid * BPW

        pltpu.sync_copy(in_hbm.at[pl.ds(b0 * S, CW)], idx_v)
        pltpu.sync_copy(pat_hbm, pat_v)

        def fuse(j, carry):
            o = j * _L
            q = lax.rem(j, S) * _L
            idx_v[pl.ds(o, _L)] = idx_v[pl.ds(o, _L)] + pat_v[pl.ds(q, _L)]
            return carry

        lax.fori_loop(0, NV, fuse, 0)

        def gstart(c, b):
            pltpu.async_copy(
                table_hbm.at[idx_v.at[pl.ds(c * GR, GR)]], rows_v.at[b], gsem[b]
            )

        def gwait(c, b):
            pltpu.make_async_copy(
                table_hbm.at[idx_v.at[pl.ds(c * GR, GR)]], rows_v.at[b], gsem[b]
            ).wait()

        def wstart(c, b):
            for i in range(_BPC):
                pltpu.async_copy(
                    rows_v.at[b, pl.ds(i * S, S)],
                    out_hbm.at[b0 + c * _BPC + i],
                    wsem[b],
                )

        def wwait(c, b):
            for i in range(_BPC):
                pltpu.make_async_copy(
                    rows_v.at[b, pl.ds(i * S, S)],
                    out_hbm.at[b0 + c * _BPC + i],
                    wsem[b],
                ).wait()

        for g in range(_K):
            gstart(g, g)

        def group(o, carry):
            base = o * _NBUF
            for b in range(_NBUF):
                c = base + b
                gwait(c, b)
                wstart(c, b)
                g = c + _K
                bg = (b + _K) % _NBUF

                @pl.when(jnp.logical_and(g < RPW, g >= _NBUF))
                def _():
                    wwait(g - _NBUF, bg)

                @pl.when(g < RPW)
                def _():
                    gstart(g, bg)

            return carry

        lax.fori_loop(0, RPW // _NBUF, group, 0)
        for b in range(_NBUF):
            wwait(RPW - _NBUF + b, b)

    return k


def kernel(inputs, table):
    B, S = inputs.shape
    V, D = table.shape
    VS = V // S
    k = _build(S, B, D, VS)
    idx_flat = inputs.reshape(-1)
    pat = jnp.tile(jnp.arange(S, dtype=jnp.int32) * VS, _L)
    return k(table, idx_flat, pat)
